# baseline (speedup 1.0000x reference)
import functools

import jax
import jax.numpy as jnp
from jax import lax
from jax.experimental import pallas as pl
from jax.experimental.pallas import tpu as pltpu
from jax.experimental.pallas import tpu_sc as plsc

B = 16384
NM, NR, NT, ND = 36, 19, 32, 16
MD, RD, TD, DD = 64, 64, 32, 32
D = 512
H = D // 2
PM_ROWS = NM * NR
PT_ROWS = NT * ND

NC, NS, L = 2, 16, 16
NW = NC * NS

B_SC = 4096
BPW = B_SC // NW
CH = 32
NCH = BPW // CH
VOC = NM + NR + NT + ND
VOCP = 128
TCB = 2048
TCB0 = B_SC // TCB
NTCB = (B - B_SC) // TCB

def _proj_body(mood_t, raga_t, tempo_t, dur_t, w, b, pm_ref, pt_ref,
               ps_ref):
    f32 = jnp.float32
    mp = jnp.dot(mood_t[...], w[0:MD, :], preferred_element_type=f32)
    rp = jnp.dot(raga_t[...], w[MD:MD + RD, :], preferred_element_type=f32)
    tp = jnp.dot(tempo_t[...], w[MD + RD:MD + RD + TD, :],
                 preferred_element_type=f32)
    dp = jnp.dot(dur_t[...], w[MD + RD + TD:, :], preferred_element_type=f32)

    def onehot(rows, cols, div, mod):
        i = lax.broadcasted_iota(jnp.int32, (rows, cols), 0)
        j = lax.broadcasted_iota(jnp.int32, (rows, cols), 1)
        k = (i // div) % mod if mod else i // div
        return (k == j).astype(f32)

    ohm = onehot(PM_ROWS, NM, NR, 0)
    ohr = onehot(PM_ROWS, NR, 1, NR)
    pm = (jnp.dot(ohm, mp, preferred_element_type=f32)
          + jnp.dot(ohr, rp, preferred_element_type=f32))
    oht = onehot(PT_ROWS, NT, ND, 0)
    ohd = onehot(PT_ROWS, ND, 1, ND)
    pt = (jnp.dot(oht, tp, preferred_element_type=f32)
          + jnp.dot(ohd, dp, preferred_element_type=f32)
          + b[...])
    pm_ref[...] = pm
    pt_ref[...] = pt
    ps_ref[...] = jnp.concatenate(
        [mp, rp, tp, dp + b[...],
         jnp.zeros((VOCP - VOC, D), f32)], axis=0).astype(jnp.bfloat16)


_proj = pl.pallas_call(
    _proj_body,
    out_shape=(
        jax.ShapeDtypeStruct((PM_ROWS, D), jnp.float32),
        jax.ShapeDtypeStruct((PT_ROWS, D), jnp.float32),
        jax.ShapeDtypeStruct((VOCP, D), jnp.bfloat16),
    ),
)


def _sc_body(mood_hbm, raga_hbm, tempo_hbm, dur_hbm, pm_hbm, pt_hbm, out_hbm,
             stage_a, stage_b, ia_v, ib_v,
             buf_a0, buf_a1, buf_b0, buf_b1, buf_o0, buf_o1,
             sga0, sga1, sgb0, sgb1, sst0, sst1):
    buf_a = (buf_a0, buf_a1)
    buf_b = (buf_b0, buf_b1)
    buf_o = (buf_o0, buf_o1)
    sga = (sga0, sga1)
    sgb = (sgb0, sgb1)
    sst = (sst0, sst1)

    wid = lax.axis_index("s") * NC + lax.axis_index("c")
    base = wid * BPW

    pltpu.sync_copy(mood_hbm.at[pl.ds(base, BPW)], stage_a)
    pltpu.sync_copy(raga_hbm.at[pl.ds(base, BPW)], stage_b)
    for k in range(BPW // L):
        sl = pl.ds(k * L, L)
        ia_v[sl] = stage_a[sl] * NR + stage_b[sl]
    pltpu.sync_copy(tempo_hbm.at[pl.ds(base, BPW)], stage_a)
    pltpu.sync_copy(dur_hbm.at[pl.ds(base, BPW)], stage_b)
    for k in range(BPW // L):
        sl = pl.ds(k * L, L)
        ib_v[sl] = stage_a[sl] * ND + stage_b[sl]

    def g_desc(c, s):
        return (pltpu.make_async_copy(pm_hbm.at[ia_v.at[pl.ds(c * CH, CH)]],
                                      buf_a[s], sga[s]),
                pltpu.make_async_copy(pt_hbm.at[ib_v.at[pl.ds(c * CH, CH)]],
                                      buf_b[s], sgb[s]))

    def s_desc(c, s):
        return pltpu.make_async_copy(
            buf_o[s], out_hbm.at[pl.ds(base + c * CH, CH)], sst[s])

    def issue_gathers(c, s):
        da, db = g_desc(c, s)
        da.start()
        db.start()

    def do_chunk(c, s, first):
        da, db = g_desc(c, s)
        da.wait()
        db.wait()

        @pl.when(jnp.logical_not(first))
        def _():
            s_desc(c - 2, s).wait()

        def row_body(r, _):
            for j in range(D // L):
                sl = pl.ds(j * L, L)
                buf_o[s][r, sl] = jnp.maximum(
                    buf_a[s][r, sl] + buf_b[s][r, sl], 0.0)
            return 0

        lax.fori_loop(0, CH, row_body, 0)
        s_desc(c, s).start()

        @pl.when(c + 2 < NCH)
        def _():
            issue_gathers(c + 2, s)

    issue_gathers(0, 0)
    issue_gathers(1, 1)

    def pair_body(i, _):
        c = i * 2
        do_chunk(c, 0, i == 0)
        do_chunk(c + 1, 1, i == 0)
        return 0

    lax.fori_loop(0, NCH // 2, pair_body, 0)
    s_desc(NCH - 2, 0).wait()
    s_desc(NCH - 1, 1).wait()


@functools.cache
def _gather():
    return pl.kernel(
        _sc_body,
        out_type=jax.ShapeDtypeStruct((B_SC, D), jnp.float32),
        mesh=plsc.VectorSubcoreMesh(core_axis_name="c", subcore_axis_name="s",
                                    num_cores=NC, num_subcores=NS),
        scratch_types=[
            pltpu.VMEM((BPW,), jnp.int32),
            pltpu.VMEM((BPW,), jnp.int32),
            pltpu.VMEM((BPW,), jnp.int32),
            pltpu.VMEM((BPW,), jnp.int32),
            pltpu.VMEM((CH, D), jnp.float32),
            pltpu.VMEM((CH, D), jnp.float32),
            pltpu.VMEM((CH, D), jnp.float32),
            pltpu.VMEM((CH, D), jnp.float32),
            pltpu.VMEM((CH, D), jnp.float32),
            pltpu.VMEM((CH, D), jnp.float32),
            pltpu.SemaphoreType.DMA,
            pltpu.SemaphoreType.DMA,
            pltpu.SemaphoreType.DMA,
            pltpu.SemaphoreType.DMA,
            pltpu.SemaphoreType.DMA,
            pltpu.SemaphoreType.DMA,
        ],
    )


def _tc_fill_body(mood_ref, raga_ref, tempo_ref, dur_ref, p_ref, out_ref):
    f32 = jnp.float32
    bf16 = jnp.bfloat16
    i0 = lax.broadcasted_iota(jnp.int32, (VOCP, TCB), 0)
    oh = ((i0 == mood_ref[...]).astype(bf16)
          + (i0 == raga_ref[...] + NM).astype(bf16)
          + (i0 == tempo_ref[...] + (NM + NR)).astype(bf16)
          + (i0 == dur_ref[...] + (NM + NR + NT)).astype(bf16))
    blk = lax.dot_general(oh, p_ref[...], (((0,), (0,)), ((), ())),
                          preferred_element_type=f32)
    out_ref[...] = jnp.maximum(blk, 0.0)


_tc_fill = pl.pallas_call(
    _tc_fill_body,
    grid=(NTCB,),
    in_specs=[
        pl.BlockSpec((1, TCB), lambda i: (0, TCB0 + i)),
        pl.BlockSpec((1, TCB), lambda i: (0, TCB0 + i)),
        pl.BlockSpec((1, TCB), lambda i: (0, TCB0 + i)),
        pl.BlockSpec((1, TCB), lambda i: (0, TCB0 + i)),
        pl.BlockSpec((VOCP, D), lambda i: (0, 0)),
    ],
    out_specs=pl.BlockSpec((TCB, D), lambda i: (TCB0 + i, 0)),
    out_shape=jax.ShapeDtypeStruct((B, D), jnp.float32),
)


def kernel(mood, raga, tempo, duration, mood_table, raga_table, tempo_table,
           duration_table, W, b):
    i32 = jnp.int32
    mood = mood.astype(i32)
    raga = raga.astype(i32)
    tempo = tempo.astype(i32)
    duration = duration.astype(i32)
    pm, pt, ps = _proj(mood_table, raga_table, tempo_table, duration_table,
                       W, b.reshape(1, D))
    out_sc = _gather()(mood, raga, tempo, duration, pm, pt)
    out_tc = _tc_fill(mood[None, :], raga[None, :], tempo[None, :],
                      duration[None, :], ps)
    return lax.dynamic_update_slice(out_tc, out_sc, (0, 0))


# speedup vs baseline: 1.0411x; 1.0411x over previous
import functools

import jax
import jax.numpy as jnp
from jax import lax
from jax.experimental import pallas as pl
from jax.experimental.pallas import tpu as pltpu
from jax.experimental.pallas import tpu_sc as plsc

B = 16384
NM, NR, NT, ND = 36, 19, 32, 16
MD, RD, TD, DD = 64, 64, 32, 32
D = 512
H = D // 2
PM_ROWS = NM * NR
PT_ROWS = NT * ND

NC, NS, L = 2, 16, 16
NW = NC * NS

B_SC = 4096
BPW = B_SC // NW
CH = 32
NCH = BPW // CH
VOC = NM + NR + NT + ND
VOCP = 128
TCB = 4096
TCB0 = B_SC // TCB
NTCB = (B - B_SC) // TCB

def _proj_body(mood_t, raga_t, tempo_t, dur_t, w, b, pm_ref, pt_ref,
               ps_ref):
    f32 = jnp.float32
    mp = jnp.dot(mood_t[...], w[0:MD, :], preferred_element_type=f32)
    rp = jnp.dot(raga_t[...], w[MD:MD + RD, :], preferred_element_type=f32)
    tp = jnp.dot(tempo_t[...], w[MD + RD:MD + RD + TD, :],
                 preferred_element_type=f32)
    dp = jnp.dot(dur_t[...], w[MD + RD + TD:, :], preferred_element_type=f32)

    def onehot(rows, cols, div, mod):
        i = lax.broadcasted_iota(jnp.int32, (rows, cols), 0)
        j = lax.broadcasted_iota(jnp.int32, (rows, cols), 1)
        k = (i // div) % mod if mod else i // div
        return (k == j).astype(f32)

    ohm = onehot(PM_ROWS, NM, NR, 0)
    ohr = onehot(PM_ROWS, NR, 1, NR)
    pm = (jnp.dot(ohm, mp, preferred_element_type=f32)
          + jnp.dot(ohr, rp, preferred_element_type=f32))
    oht = onehot(PT_ROWS, NT, ND, 0)
    ohd = onehot(PT_ROWS, ND, 1, ND)
    pt = (jnp.dot(oht, tp, preferred_element_type=f32)
          + jnp.dot(ohd, dp, preferred_element_type=f32)
          + b[...])
    pm_ref[...] = pm
    pt_ref[...] = pt
    ps_ref[...] = jnp.concatenate(
        [mp, rp, tp, dp + b[...],
         jnp.zeros((VOCP - VOC, D), f32)], axis=0).astype(jnp.bfloat16)


_proj = pl.pallas_call(
    _proj_body,
    out_shape=(
        jax.ShapeDtypeStruct((PM_ROWS, D), jnp.float32),
        jax.ShapeDtypeStruct((PT_ROWS, D), jnp.float32),
        jax.ShapeDtypeStruct((VOCP, D), jnp.bfloat16),
    ),
)


def _sc_body(mood_hbm, raga_hbm, tempo_hbm, dur_hbm, pm_hbm, pt_hbm, out_hbm,
             stage_a, stage_b, ia_v, ib_v,
             buf_a0, buf_a1, buf_b0, buf_b1, buf_o0, buf_o1,
             sga0, sga1, sgb0, sgb1, sst0, sst1):
    buf_a = (buf_a0, buf_a1)
    buf_b = (buf_b0, buf_b1)
    buf_o = (buf_o0, buf_o1)
    sga = (sga0, sga1)
    sgb = (sgb0, sgb1)
    sst = (sst0, sst1)

    wid = lax.axis_index("s") * NC + lax.axis_index("c")
    base = wid * BPW

    pltpu.sync_copy(mood_hbm.at[pl.ds(base, BPW)], stage_a)
    pltpu.sync_copy(raga_hbm.at[pl.ds(base, BPW)], stage_b)
    for k in range(BPW // L):
        sl = pl.ds(k * L, L)
        ia_v[sl] = stage_a[sl] * NR + stage_b[sl]
    pltpu.sync_copy(tempo_hbm.at[pl.ds(base, BPW)], stage_a)
    pltpu.sync_copy(dur_hbm.at[pl.ds(base, BPW)], stage_b)
    for k in range(BPW // L):
        sl = pl.ds(k * L, L)
        ib_v[sl] = stage_a[sl] * ND + stage_b[sl]

    def g_desc(c, s):
        return (pltpu.make_async_copy(pm_hbm.at[ia_v.at[pl.ds(c * CH, CH)]],
                                      buf_a[s], sga[s]),
                pltpu.make_async_copy(pt_hbm.at[ib_v.at[pl.ds(c * CH, CH)]],
                                      buf_b[s], sgb[s]))

    def s_desc(c, s):
        return pltpu.make_async_copy(
            buf_o[s], out_hbm.at[pl.ds(base + c * CH, CH)], sst[s])

    def issue_gathers(c, s):
        da, db = g_desc(c, s)
        da.start()
        db.start()

    def do_chunk(c, s, first):
        da, db = g_desc(c, s)
        da.wait()
        db.wait()

        @pl.when(jnp.logical_not(first))
        def _():
            s_desc(c - 2, s).wait()

        def row_body(r, _):
            for j in range(D // L):
                sl = pl.ds(j * L, L)
                buf_o[s][r, sl] = jnp.maximum(
                    buf_a[s][r, sl] + buf_b[s][r, sl], 0.0)
            return 0

        lax.fori_loop(0, CH, row_body, 0)
        s_desc(c, s).start()

        @pl.when(c + 2 < NCH)
        def _():
            issue_gathers(c + 2, s)

    issue_gathers(0, 0)
    issue_gathers(1, 1)

    def pair_body(i, _):
        c = i * 2
        do_chunk(c, 0, i == 0)
        do_chunk(c + 1, 1, i == 0)
        return 0

    lax.fori_loop(0, NCH // 2, pair_body, 0)
    s_desc(NCH - 2, 0).wait()
    s_desc(NCH - 1, 1).wait()


@functools.cache
def _gather():
    return pl.kernel(
        _sc_body,
        out_type=jax.ShapeDtypeStruct((B, D), jnp.float32),
        mesh=plsc.VectorSubcoreMesh(core_axis_name="c", subcore_axis_name="s",
                                    num_cores=NC, num_subcores=NS),
        scratch_types=[
            pltpu.VMEM((BPW,), jnp.int32),
            pltpu.VMEM((BPW,), jnp.int32),
            pltpu.VMEM((BPW,), jnp.int32),
            pltpu.VMEM((BPW,), jnp.int32),
            pltpu.VMEM((CH, D), jnp.float32),
            pltpu.VMEM((CH, D), jnp.float32),
            pltpu.VMEM((CH, D), jnp.float32),
            pltpu.VMEM((CH, D), jnp.float32),
            pltpu.VMEM((CH, D), jnp.float32),
            pltpu.VMEM((CH, D), jnp.float32),
            pltpu.SemaphoreType.DMA,
            pltpu.SemaphoreType.DMA,
            pltpu.SemaphoreType.DMA,
            pltpu.SemaphoreType.DMA,
            pltpu.SemaphoreType.DMA,
            pltpu.SemaphoreType.DMA,
        ],
    )


def _tc_fill_body(mood_ref, raga_ref, tempo_ref, dur_ref, p_ref, _outsc,
                  out_ref):
    f32 = jnp.float32
    bf16 = jnp.bfloat16
    i0 = lax.broadcasted_iota(jnp.int32, (VOCP, TCB), 0)
    oh = ((i0 == mood_ref[...]).astype(bf16)
          + (i0 == raga_ref[...] + NM).astype(bf16)
          + (i0 == tempo_ref[...] + (NM + NR)).astype(bf16)
          + (i0 == dur_ref[...] + (NM + NR + NT)).astype(bf16))
    blk = lax.dot_general(oh, p_ref[...], (((0,), (0,)), ((), ())),
                          preferred_element_type=f32)
    out_ref[...] = jnp.maximum(blk, 0.0)


_tc_fill = pl.pallas_call(
    _tc_fill_body,
    grid=(NTCB,),
    in_specs=[
        pl.BlockSpec((1, TCB), lambda i: (0, TCB0 + i)),
        pl.BlockSpec((1, TCB), lambda i: (0, TCB0 + i)),
        pl.BlockSpec((1, TCB), lambda i: (0, TCB0 + i)),
        pl.BlockSpec((1, TCB), lambda i: (0, TCB0 + i)),
        pl.BlockSpec((VOCP, D), lambda i: (0, 0)),
        pl.BlockSpec(memory_space=pl.ANY),
    ],
    out_specs=pl.BlockSpec((TCB, D), lambda i: (TCB0 + i, 0)),
    out_shape=jax.ShapeDtypeStruct((B, D), jnp.float32),
    input_output_aliases={5: 0},
)


def kernel(mood, raga, tempo, duration, mood_table, raga_table, tempo_table,
           duration_table, W, b):
    i32 = jnp.int32
    mood = mood.astype(i32)
    raga = raga.astype(i32)
    tempo = tempo.astype(i32)
    duration = duration.astype(i32)
    pm, pt, ps = _proj(mood_table, raga_table, tempo_table, duration_table,
                       W, b.reshape(1, D))
    out_sc = _gather()(mood, raga, tempo, duration, pm, pt)
    return _tc_fill(mood[None, :], raga[None, :], tempo[None, :],
                    duration[None, :], ps, out_sc)


# speedup vs baseline: 1.0486x; 1.0071x over previous
import functools

import jax
import jax.numpy as jnp
from jax import lax
from jax.experimental import pallas as pl
from jax.experimental.pallas import tpu as pltpu
from jax.experimental.pallas import tpu_sc as plsc

B = 16384
NM, NR, NT, ND = 36, 19, 32, 16
MD, RD, TD, DD = 64, 64, 32, 32
D = 512
PM_ROWS = NM * NR
PT_ROWS = NT * ND

NC, NS, L = 2, 16, 16
NW = NC * NS

B_SC = 4096
BPW = B_SC // NW
CH = 32
NCH = BPW // CH
VOC = NM + NR + NT + ND
VOCP = 128
TCB = 2048
TCB0 = B_SC // TCB
NTCB = (B - B_SC) // TCB

def _proj_body(mood_t, raga_t, tempo_t, dur_t, w, b, pm_ref, pt_ref,
               ps_ref):
    f32 = jnp.float32
    mp = jnp.dot(mood_t[...], w[0:MD, :], preferred_element_type=f32)
    rp = jnp.dot(raga_t[...], w[MD:MD + RD, :], preferred_element_type=f32)
    tp = jnp.dot(tempo_t[...], w[MD + RD:MD + RD + TD, :],
                 preferred_element_type=f32)
    dp = jnp.dot(dur_t[...], w[MD + RD + TD:, :], preferred_element_type=f32)

    def onehot(rows, cols, div, mod):
        i = lax.broadcasted_iota(jnp.int32, (rows, cols), 0)
        j = lax.broadcasted_iota(jnp.int32, (rows, cols), 1)
        k = (i // div) % mod if mod else i // div
        return (k == j).astype(f32)

    ohm = onehot(PM_ROWS, NM, NR, 0)
    ohr = onehot(PM_ROWS, NR, 1, NR)
    pm = (jnp.dot(ohm, mp, preferred_element_type=f32)
          + jnp.dot(ohr, rp, preferred_element_type=f32))
    oht = onehot(PT_ROWS, NT, ND, 0)
    ohd = onehot(PT_ROWS, ND, 1, ND)
    pt = (jnp.dot(oht, tp, preferred_element_type=f32)
          + jnp.dot(ohd, dp, preferred_element_type=f32)
          + b[...])
    pm_ref[...] = pm
    pt_ref[...] = pt
    ps_ref[...] = jnp.concatenate(
        [mp, rp, tp, dp + b[...],
         jnp.zeros((VOCP - VOC, D), f32)], axis=0).astype(jnp.bfloat16)


_proj = pl.pallas_call(
    _proj_body,
    out_shape=(
        jax.ShapeDtypeStruct((PM_ROWS, D), jnp.float32),
        jax.ShapeDtypeStruct((PT_ROWS, D), jnp.float32),
        jax.ShapeDtypeStruct((VOCP, D), jnp.bfloat16),
    ),
)


def _sc_body(mood_hbm, raga_hbm, tempo_hbm, dur_hbm, pm_hbm, pt_hbm, out_hbm,
             stage_a, stage_b, ia_v, ib_v,
             buf_a0, buf_a1, buf_b0, buf_b1, buf_o0, buf_o1,
             sga0, sga1, sgb0, sgb1, sst0, sst1):
    buf_a = (buf_a0, buf_a1)
    buf_b = (buf_b0, buf_b1)
    buf_o = (buf_o0, buf_o1)
    sga = (sga0, sga1)
    sgb = (sgb0, sgb1)
    sst = (sst0, sst1)

    wid = lax.axis_index("s") * NC + lax.axis_index("c")
    base = wid * BPW

    pltpu.sync_copy(mood_hbm.at[pl.ds(base, BPW)], stage_a)
    pltpu.sync_copy(raga_hbm.at[pl.ds(base, BPW)], stage_b)
    for k in range(BPW // L):
        sl = pl.ds(k * L, L)
        ia_v[sl] = stage_a[sl] * NR + stage_b[sl]
    pltpu.sync_copy(tempo_hbm.at[pl.ds(base, BPW)], stage_a)
    pltpu.sync_copy(dur_hbm.at[pl.ds(base, BPW)], stage_b)
    for k in range(BPW // L):
        sl = pl.ds(k * L, L)
        ib_v[sl] = stage_a[sl] * ND + stage_b[sl]

    def g_desc(c, s):
        return (pltpu.make_async_copy(pm_hbm.at[ia_v.at[pl.ds(c * CH, CH)]],
                                      buf_a[s], sga[s]),
                pltpu.make_async_copy(pt_hbm.at[ib_v.at[pl.ds(c * CH, CH)]],
                                      buf_b[s], sgb[s]))

    def s_desc(c, s):
        return pltpu.make_async_copy(
            buf_o[s], out_hbm.at[pl.ds(base + c * CH, CH)], sst[s])

    def issue_gathers(c, s):
        da, db = g_desc(c, s)
        da.start()
        db.start()

    def do_chunk(c, s, first):
        da, db = g_desc(c, s)
        da.wait()
        db.wait()

        @pl.when(jnp.logical_not(first))
        def _():
            s_desc(c - 2, s).wait()

        def row_body(r, _):
            for j in range(D // L):
                sl = pl.ds(j * L, L)
                buf_o[s][r, sl] = jnp.maximum(
                    buf_a[s][r, sl] + buf_b[s][r, sl], 0.0)
            return 0

        lax.fori_loop(0, CH, row_body, 0)
        s_desc(c, s).start()

        @pl.when(c + 2 < NCH)
        def _():
            issue_gathers(c + 2, s)

    issue_gathers(0, 0)
    issue_gathers(1, 1)

    def pair_body(i, _):
        c = i * 2
        do_chunk(c, 0, i == 0)
        do_chunk(c + 1, 1, i == 0)
        return 0

    lax.fori_loop(0, NCH // 2, pair_body, 0)
    s_desc(NCH - 2, 0).wait()
    s_desc(NCH - 1, 1).wait()


@functools.cache
def _gather():
    return pl.kernel(
        _sc_body,
        out_type=jax.ShapeDtypeStruct((B, D), jnp.float32),
        mesh=plsc.VectorSubcoreMesh(core_axis_name="c", subcore_axis_name="s",
                                    num_cores=NC, num_subcores=NS),
        scratch_types=[
            pltpu.VMEM((BPW,), jnp.int32),
            pltpu.VMEM((BPW,), jnp.int32),
            pltpu.VMEM((BPW,), jnp.int32),
            pltpu.VMEM((BPW,), jnp.int32),
            pltpu.VMEM((CH, D), jnp.float32),
            pltpu.VMEM((CH, D), jnp.float32),
            pltpu.VMEM((CH, D), jnp.float32),
            pltpu.VMEM((CH, D), jnp.float32),
            pltpu.VMEM((CH, D), jnp.float32),
            pltpu.VMEM((CH, D), jnp.float32),
            pltpu.SemaphoreType.DMA,
            pltpu.SemaphoreType.DMA,
            pltpu.SemaphoreType.DMA,
            pltpu.SemaphoreType.DMA,
            pltpu.SemaphoreType.DMA,
            pltpu.SemaphoreType.DMA,
        ],
    )


def _tc_fill_body(mood_ref, raga_ref, tempo_ref, dur_ref, p_ref, _outsc,
                  out_ref):
    f32 = jnp.float32
    bf16 = jnp.bfloat16
    i0 = lax.broadcasted_iota(jnp.int32, (VOCP, TCB), 0)
    oh = ((i0 == mood_ref[...]).astype(bf16)
          + (i0 == raga_ref[...] + NM).astype(bf16)
          + (i0 == tempo_ref[...] + (NM + NR)).astype(bf16)
          + (i0 == dur_ref[...] + (NM + NR + NT)).astype(bf16))
    blk = lax.dot_general(oh, p_ref[...], (((0,), (0,)), ((), ())),
                          preferred_element_type=f32)
    out_ref[...] = jnp.maximum(blk, 0.0)


_tc_fill = pl.pallas_call(
    _tc_fill_body,
    grid=(NTCB,),
    in_specs=[
        pl.BlockSpec((1, TCB), lambda i: (0, TCB0 + i)),
        pl.BlockSpec((1, TCB), lambda i: (0, TCB0 + i)),
        pl.BlockSpec((1, TCB), lambda i: (0, TCB0 + i)),
        pl.BlockSpec((1, TCB), lambda i: (0, TCB0 + i)),
        pl.BlockSpec((VOCP, D), lambda i: (0, 0)),
        pl.BlockSpec(memory_space=pl.ANY),
    ],
    out_specs=pl.BlockSpec((TCB, D), lambda i: (TCB0 + i, 0)),
    out_shape=jax.ShapeDtypeStruct((B, D), jnp.float32),
    input_output_aliases={5: 0},
)


def kernel(mood, raga, tempo, duration, mood_table, raga_table, tempo_table,
           duration_table, W, b):
    i32 = jnp.int32
    mood = mood.astype(i32)
    raga = raga.astype(i32)
    tempo = tempo.astype(i32)
    duration = duration.astype(i32)
    pm, pt, ps = _proj(mood_table, raga_table, tempo_table, duration_table,
                       W, b.reshape(1, D))
    out_sc = _gather()(mood, raga, tempo, duration, pm, pt)
    return _tc_fill(mood[None, :], raga[None, :], tempo[None, :],
                    duration[None, :], ps, out_sc)
